# baseline (speedup 1.0000x reference)
import functools

import jax
import jax.numpy as jnp
import numpy as np
from jax import lax
from jax.experimental import pallas as pl
from jax.experimental.pallas import tpu as pltpu
from jax.experimental.pallas import tpu_sc as plsc

B = 16384
D = 32
OUT_COLS = 228
NUM_BUCKETS = 1000

_info = plsc.get_sparse_core_info()
NC = _info.num_cores
NS = _info.num_subcores
L = _info.num_lanes
NW = NC * NS
B_PER_W = B // NW
CHUNK = 32
N_CHUNKS = B_PER_W // CHUNK

_DEN = float(np.sqrt(np.float32(1.0 / 12.0 + 1e-7)))
_FIELD_COLS = (0, 32, 64, 96, 129, 162, 195)
_NORM_COLS = (128, 161, 194, 227)

NUM_USERS = 1000000
_BLKC = 2048
_TGRID = (NUM_USERS + _BLKC - 1) // _BLKC


def _transpose_body(in_ref, out_ref):
    x = in_ref[...]
    y = x.reshape(32, _BLKC // 4, 4)
    z = jnp.transpose(y, (1, 2, 0))
    out_ref[...] = z.reshape(_BLKC // 4, 128)


def _relayout_user(ut_t):
    return pl.pallas_call(
        _transpose_body,
        grid=(_TGRID,),
        in_specs=[pl.BlockSpec((32, _BLKC), lambda i: (0, i))],
        out_specs=pl.BlockSpec((_BLKC // 4, 128), lambda i: (i, 0)),
        out_shape=jax.ShapeDtypeStruct((NUM_USERS // 4, 128), jnp.float32),
    )(ut_t)


def _body(uid_h, g_h, s_h, rd_h, hy_h, vt_h, fv_h,
          ut_h, gs_h, bk_h, bnd_h,
          out_h,
          bnd_v, uid_v, gv_v, sv_v, uq_v, ucol_v, gsidx_v, xb_v,
          bidx0, bidx1, bidx2, bidx3, nrm_v,
          ud0, ud1, gsd0, gsd1,
          bk00, bk01, bk02, bk03, bk10, bk11, bk12, bk13,
          stag0, stag1,
          sem_g0, sem_g1, sem_o0, sem_o1):
    wid = lax.axis_index("s") * NC + lax.axis_index("c")
    base = wid * B_PER_W
    tsl = pl.ds(base, B_PER_W)
    pltpu.sync_copy(bnd_h, bnd_v)
    pltpu.sync_copy(uid_h.at[tsl], uid_v)
    pltpu.sync_copy(g_h.at[tsl], gv_v)
    pltpu.sync_copy(s_h.at[tsl], sv_v)
    pltpu.sync_copy(rd_h.at[tsl], xb_v.at[0])
    pltpu.sync_copy(hy_h.at[tsl], xb_v.at[1])
    pltpu.sync_copy(vt_h.at[tsl], xb_v.at[2])
    pltpu.sync_copy(fv_h.at[tsl], xb_v.at[3])

    bidx = (bidx0, bidx1, bidx2, bidx3)

    def _precompute(g, carry):
        s16 = pl.ds(g * L, L)
        uid = uid_v[s16]
        uq_v[s16] = lax.shift_right_logical(uid, 2)
        ucol_v[s16] = (uid & 3) * D
        gsidx_v[s16] = gv_v[s16] * 8 + sv_v[s16]
        for f in range(4):
            x = xb_v[f, s16]
            c = x * np.float32(NUM_BUCKETS - 1)
            t = c.astype(jnp.int32)
            e = t + jnp.where(t.astype(jnp.float32) < c, 1, 0)
            p = jnp.minimum(jnp.maximum(e - 2, 0), NUM_BUCKETS - 4)
            cnt = jnp.where(plsc.load_gather(bnd_v, [p]) < x, 1, 0)
            for k in range(1, 4):
                cnt = cnt + jnp.where(plsc.load_gather(bnd_v, [p + k]) < x, 1, 0)
            bidx[f][s16] = p + cnt
            nrm_v[f, s16] = (x - np.float32(0.5)) / np.float32(_DEN)
        return carry
    lax.fori_loop(0, B_PER_W // L, _precompute, 0)

    gsems = (sem_g0, sem_g1)
    uds = (ud0, ud1)
    gsds = (gsd0, gsd1)
    bkds = ((bk00, bk01, bk02, bk03), (bk10, bk11, bk12, bk13))
    stags = (stag0, stag1)
    osems = (sem_o0, sem_o1)

    def _fire(ci, s):
        csl = pl.ds(ci * CHUNK, CHUNK)
        sem = gsems[s]
        cps = [
            pltpu.async_copy(ut_h.at[uq_v.at[csl]], uds[s], sem),
            pltpu.async_copy(gs_h.at[gsidx_v.at[csl]], gsds[s], sem),
        ]
        for f in range(4):
            cps.append(
                pltpu.async_copy(bk_h.at[bidx[f].at[csl]], bkds[s][f], sem))
        return cps

    pend_g = {0: _fire(0, 0)}
    pend_o = {}
    for ci in range(N_CHUNKS):
        s = ci % 2
        if ci + 1 < N_CHUNKS:
            pend_g[ci + 1] = _fire(ci + 1, 1 - s)
        for cp in pend_g.pop(ci):
            cp.wait()
        if (ci - 2) in pend_o:
            pend_o.pop(ci - 2).wait()

        cb0 = ci * CHUNK
        ud_s, gsd_s, bkd_s, stag_s = uds[s], gsds[s], bkds[s], stags[s]

        def _repack(r, inner):
            rb = r * OUT_COLS
            cl = ucol_v[pl.ds(cb0 + r, L)][0]
            stag_s[pl.ds(rb, L)] = ud_s[r, pl.ds(cl, L)]
            stag_s[pl.ds(rb + L, L)] = ud_s[r, pl.ds(cl + L, L)]
            for c in range(0, 2 * D, L):
                stag_s[pl.ds(rb + 32 + c, L)] = gsd_s[r, pl.ds(c, L)]
            for f in range(4):
                col = _FIELD_COLS[3 + f]
                for c in range(0, D, L):
                    stag_s[pl.ds(rb + col + c, L)] = \
                        bkd_s[f][r, pl.ds(f * D + c, L)]
            return inner
        lax.fori_loop(0, CHUNK, _repack, 0)

        for f in range(4):
            for j in range(CHUNK // L):
                rows = jnp.arange(j * L, (j + 1) * L, dtype=jnp.int32)
                pos = rows * OUT_COLS + _NORM_COLS[f]
                nx = nrm_v[f, pl.ds(cb0 + j * L, L)]
                plsc.store_scatter(stag_s, [pos], nx)

        cb = base + cb0
        pend_o[ci] = pltpu.async_copy(
            stag_s, out_h.at[pl.ds(cb * OUT_COLS, CHUNK * OUT_COLS)],
            osems[s])
    for cp in pend_o.values():
        cp.wait()


def kernel(user_id, gender, status, regis_date, history, voting, favourite,
           user_table, gender_table, status_table,
           rgst_table, hsty_table, vote_table, favr_table):
    bnd = jnp.linspace(0.0, 1.0, NUM_BUCKETS).astype(jnp.float32)
    bnd = jnp.concatenate([bnd, jnp.full((8,), 2.0, jnp.float32)])
    ut2 = _relayout_user(user_table.T)
    gs = jnp.concatenate([jnp.repeat(gender_table, 8, axis=0),
                          jnp.tile(status_table, (3, 1)),
                          jnp.zeros((24, 2 * D), jnp.float32)], axis=1)
    bk = jnp.concatenate([rgst_table, hsty_table, vote_table, favr_table],
                         axis=1)
    mesh = plsc.VectorSubcoreMesh(core_axis_name="c", subcore_axis_name="s")
    gd = [pltpu.VMEM((CHUNK, 4 * D), jnp.float32)] * 12
    run = functools.partial(
        pl.kernel, mesh=mesh,
        compiler_params=pltpu.CompilerParams(needs_layout_passes=False),
        out_type=jax.ShapeDtypeStruct((B * OUT_COLS,), jnp.float32),
        scratch_types=[
            pltpu.VMEM((NUM_BUCKETS + 8,), jnp.float32),
            pltpu.VMEM((B_PER_W,), jnp.int32),
            pltpu.VMEM((B_PER_W,), jnp.int32),
            pltpu.VMEM((B_PER_W,), jnp.int32),
            pltpu.VMEM((B_PER_W,), jnp.int32),
            pltpu.VMEM((B_PER_W + L,), jnp.int32),
            pltpu.VMEM((B_PER_W,), jnp.int32),
            pltpu.VMEM((4, B_PER_W), jnp.float32),
            pltpu.VMEM((B_PER_W,), jnp.int32),
            pltpu.VMEM((B_PER_W,), jnp.int32),
            pltpu.VMEM((B_PER_W,), jnp.int32),
            pltpu.VMEM((B_PER_W,), jnp.int32),
            pltpu.VMEM((4, B_PER_W), jnp.float32),
        ] + gd + [
            pltpu.VMEM((CHUNK * OUT_COLS,), jnp.float32),
            pltpu.VMEM((CHUNK * OUT_COLS,), jnp.float32),
            pltpu.SemaphoreType.DMA,
            pltpu.SemaphoreType.DMA,
            pltpu.SemaphoreType.DMA,
            pltpu.SemaphoreType.DMA,
        ],
    )(_body)
    flat = run(user_id.astype(jnp.int32), gender.astype(jnp.int32),
               status.astype(jnp.int32), regis_date, history, voting,
               favourite, ut2, gs, bk, bnd)
    return flat.reshape(B, OUT_COLS)


# speedup vs baseline: 4.1418x; 4.1418x over previous
import functools

import jax
import jax.numpy as jnp
import numpy as np
from jax import lax
from jax.experimental import pallas as pl
from jax.experimental.pallas import tpu as pltpu
from jax.experimental.pallas import tpu_sc as plsc

B = 16384
D = 32
OUT_COLS = 228
NUM_BUCKETS = 1000

_info = plsc.get_sparse_core_info()
NC = _info.num_cores
NS = _info.num_subcores
L = _info.num_lanes
NW = NC * NS
B_PER_W = B // NW
CHUNK = 32
N_CHUNKS = B_PER_W // CHUNK

_DEN = float(np.sqrt(np.float32(1.0 / 12.0 + 1e-7)))
_FIELD_COLS = (0, 32, 64, 96, 129, 162, 195)
_NORM_COLS = (128, 161, 194, 227)

NUM_USERS = 1000000
_BLKC = 2048
_TGRID = (NUM_USERS + _BLKC - 1) // _BLKC


def _transpose_body(in_ref, out_ref):
    x = in_ref[...]
    y = x.reshape(32, _BLKC // 4, 4)
    z = jnp.transpose(y, (1, 2, 0))
    out_ref[...] = z.reshape(_BLKC // 4, 128)


def _relayout_user(ut_t):
    return pl.pallas_call(
        _transpose_body,
        grid=(_TGRID,),
        in_specs=[pl.BlockSpec((32, _BLKC), lambda i: (0, i))],
        out_specs=pl.BlockSpec((_BLKC // 4, 128), lambda i: (i, 0)),
        out_shape=jax.ShapeDtypeStruct((NUM_USERS // 4, 128), jnp.float32),
    )(ut_t)


def _body(uid_h, g_h, s_h, rd_h, hy_h, vt_h, fv_h,
          ut_h, gs_h, bk_h, bnd_h,
          out_h,
          bnd_v, uid_v, gv_v, sv_v, uq_v, ucol_v, gsidx_v, xb_v,
          bidx0, bidx1, bidx2, bidx3, nrm_v,
          ud0, ud1, gsd0, gsd1,
          bk00, bk01, bk02, bk03, bk10, bk11, bk12, bk13,
          stag0, stag1,
          sem_g0, sem_g1, sem_o0, sem_o1):
    wid = lax.axis_index("s") * NC + lax.axis_index("c")
    base = wid * B_PER_W
    tsl = pl.ds(base, B_PER_W)
    pltpu.sync_copy(bnd_h, bnd_v)
    pltpu.sync_copy(uid_h.at[tsl], uid_v)
    pltpu.sync_copy(g_h.at[tsl], gv_v)
    pltpu.sync_copy(s_h.at[tsl], sv_v)
    pltpu.sync_copy(rd_h.at[tsl], xb_v.at[0])
    pltpu.sync_copy(hy_h.at[tsl], xb_v.at[1])
    pltpu.sync_copy(vt_h.at[tsl], xb_v.at[2])
    pltpu.sync_copy(fv_h.at[tsl], xb_v.at[3])

    bidx = (bidx0, bidx1, bidx2, bidx3)

    def _precompute(g, carry):
        s16 = pl.ds(g * L, L)
        uid = uid_v[s16]
        uq_v[s16] = lax.shift_right_logical(uid, 2)
        ucol_v[s16] = (uid & 3) * D
        gsidx_v[s16] = gv_v[s16] * 8 + sv_v[s16]
        for f in range(4):
            x = xb_v[f, s16]
            c = x * np.float32(NUM_BUCKETS - 1)
            t = c.astype(jnp.int32)
            e = t + jnp.where(t.astype(jnp.float32) < c, 1, 0)
            p = jnp.minimum(jnp.maximum(e - 2, 0), NUM_BUCKETS - 4)
            cnt = jnp.where(plsc.load_gather(bnd_v, [p]) < x, 1, 0)
            for k in range(1, 4):
                cnt = cnt + jnp.where(plsc.load_gather(bnd_v, [p + k]) < x, 1, 0)
            bidx[f][s16] = p + cnt
            nrm_v[f, s16] = (x - np.float32(0.5)) / np.float32(_DEN)
        return carry
    lax.fori_loop(0, B_PER_W // L, _precompute, 0)

    gsems = (sem_g0, sem_g1)
    uds = (ud0, ud1)
    gsds = (gsd0, gsd1)
    bkds = ((bk00, bk01, bk02, bk03), (bk10, bk11, bk12, bk13))
    stags = (stag0, stag1)
    osems = (sem_o0, sem_o1)

    def _fire(ci, s):
        csl = pl.ds(ci * CHUNK, CHUNK)
        sem = gsems[s]
        cps = [
            pltpu.async_copy(ut_h.at[uq_v.at[csl]], uds[s], sem),
            pltpu.async_copy(gs_h.at[gsidx_v.at[csl]], gsds[s], sem),
        ]
        for f in range(4):
            cps.append(
                pltpu.async_copy(bk_h.at[bidx[f].at[csl]], bkds[s][f], sem))
        return cps

    pend_g = {0: _fire(0, 0)}
    pend_o = {}
    for ci in range(N_CHUNKS):
        s = ci % 2
        if ci + 1 < N_CHUNKS:
            pend_g[ci + 1] = _fire(ci + 1, 1 - s)
        for cp in pend_g.pop(ci):
            cp.wait()
        if (ci - 2) in pend_o:
            pend_o.pop(ci - 2).wait()

        cb0 = ci * CHUNK
        ud_s, gsd_s, bkd_s, stag_s = uds[s], gsds[s], bkds[s], stags[s]

        def _repack(r, inner):
            rb = r * OUT_COLS
            cl = ucol_v[pl.ds(cb0 + r, L)][0]
            stag_s[pl.ds(rb, L)] = ud_s[r, pl.ds(cl, L)]
            stag_s[pl.ds(rb + L, L)] = ud_s[r, pl.ds(cl + L, L)]
            for c in range(0, 2 * D, L):
                stag_s[pl.ds(rb + 32 + c, L)] = gsd_s[r, pl.ds(c, L)]
            for f in range(4):
                col = _FIELD_COLS[3 + f]
                for c in range(0, D, L):
                    stag_s[pl.ds(rb + col + c, L)] = \
                        bkd_s[f][r, pl.ds(f * D + c, L)]
            return inner
        lax.fori_loop(0, CHUNK, _repack, 0)

        for f in range(4):
            for j in range(CHUNK // L):
                rows = jnp.arange(j * L, (j + 1) * L, dtype=jnp.int32)
                pos = rows * OUT_COLS + _NORM_COLS[f]
                nx = nrm_v[f, pl.ds(cb0 + j * L, L)]
                plsc.store_scatter(stag_s, [pos], nx)

        cb = base + cb0
        pend_o[ci] = pltpu.async_copy(
            stag_s, out_h.at[pl.ds(cb * OUT_COLS, CHUNK * OUT_COLS)],
            osems[s])
    for cp in pend_o.values():
        cp.wait()


def kernel(user_id, gender, status, regis_date, history, voting, favourite,
           user_table, gender_table, status_table,
           rgst_table, hsty_table, vote_table, favr_table):
    bnd = jnp.linspace(0.0, 1.0, NUM_BUCKETS).astype(jnp.float32)
    bnd = jnp.concatenate([bnd, jnp.full((8,), 2.0, jnp.float32)])
    ut2 = user_table.reshape(NUM_USERS // 4, 4 * D)
    gs = jnp.concatenate([jnp.repeat(gender_table, 8, axis=0),
                          jnp.tile(status_table, (3, 1)),
                          jnp.zeros((24, 2 * D), jnp.float32)], axis=1)
    bk = jnp.concatenate([rgst_table, hsty_table, vote_table, favr_table],
                         axis=1)
    mesh = plsc.VectorSubcoreMesh(core_axis_name="c", subcore_axis_name="s")
    gd = [pltpu.VMEM((CHUNK, 4 * D), jnp.float32)] * 12
    run = functools.partial(
        pl.kernel, mesh=mesh,
        compiler_params=pltpu.CompilerParams(needs_layout_passes=False),
        out_type=jax.ShapeDtypeStruct((B * OUT_COLS,), jnp.float32),
        scratch_types=[
            pltpu.VMEM((NUM_BUCKETS + 8,), jnp.float32),
            pltpu.VMEM((B_PER_W,), jnp.int32),
            pltpu.VMEM((B_PER_W,), jnp.int32),
            pltpu.VMEM((B_PER_W,), jnp.int32),
            pltpu.VMEM((B_PER_W,), jnp.int32),
            pltpu.VMEM((B_PER_W + L,), jnp.int32),
            pltpu.VMEM((B_PER_W,), jnp.int32),
            pltpu.VMEM((4, B_PER_W), jnp.float32),
            pltpu.VMEM((B_PER_W,), jnp.int32),
            pltpu.VMEM((B_PER_W,), jnp.int32),
            pltpu.VMEM((B_PER_W,), jnp.int32),
            pltpu.VMEM((B_PER_W,), jnp.int32),
            pltpu.VMEM((4, B_PER_W), jnp.float32),
        ] + gd + [
            pltpu.VMEM((CHUNK * OUT_COLS,), jnp.float32),
            pltpu.VMEM((CHUNK * OUT_COLS,), jnp.float32),
            pltpu.SemaphoreType.DMA,
            pltpu.SemaphoreType.DMA,
            pltpu.SemaphoreType.DMA,
            pltpu.SemaphoreType.DMA,
        ],
    )(_body)
    flat = run(user_id.astype(jnp.int32), gender.astype(jnp.int32),
               status.astype(jnp.int32), regis_date, history, voting,
               favourite, ut2, gs, bk, bnd)
    return flat.reshape(B, OUT_COLS)


# speedup vs baseline: 4.2322x; 1.0218x over previous
import functools

import jax
import jax.numpy as jnp
import numpy as np
from jax import lax
from jax.experimental import pallas as pl
from jax.experimental.pallas import tpu as pltpu
from jax.experimental.pallas import tpu_sc as plsc

B = 16384
D = 32
OUT_COLS = 228
NUM_BUCKETS = 1000

_info = plsc.get_sparse_core_info()
NC = _info.num_cores
NS = _info.num_subcores
L = _info.num_lanes
NW = NC * NS
B_PER_W = B // NW
CHUNK = 32
N_CHUNKS = B_PER_W // CHUNK

_DEN = float(np.sqrt(np.float32(1.0 / 12.0 + 1e-7)))
_FIELD_COLS = (0, 32, 64, 96, 129, 162, 195)
_NORM_COLS = (128, 161, 194, 227)

NUM_USERS = 1000000
_BLKC = 2048
_TGRID = (NUM_USERS + _BLKC - 1) // _BLKC


def _transpose_body(in_ref, out_ref):
    x = in_ref[...]
    y = x.reshape(32, _BLKC // 4, 4)
    z = jnp.transpose(y, (1, 2, 0))
    out_ref[...] = z.reshape(_BLKC // 4, 128)


def _relayout_user(ut_t):
    return pl.pallas_call(
        _transpose_body,
        grid=(_TGRID,),
        in_specs=[pl.BlockSpec((32, _BLKC), lambda i: (0, i))],
        out_specs=pl.BlockSpec((_BLKC // 4, 128), lambda i: (i, 0)),
        out_shape=jax.ShapeDtypeStruct((NUM_USERS // 4, 128), jnp.float32),
    )(ut_t)


def _body(uid_h, g_h, s_h, rd_h, hy_h, vt_h, fv_h,
          ut_h, gs_h, bk_h, bnd_h,
          out_h,
          bnd_v, uid_v, gv_v, sv_v, uq_v, ucol_v, gsidx_v, xb_v,
          bidx0, bidx1, bidx2, bidx3, nrm_v,
          ud0, ud1, gsd0, gsd1,
          bk00, bk01, bk02, bk03, bk10, bk11, bk12, bk13,
          stag0, stag1,
          sem_g0, sem_g1, sem_o0, sem_o1):
    wid = lax.axis_index("s") * NC + lax.axis_index("c")
    base = wid * B_PER_W
    tsl = pl.ds(base, B_PER_W)
    pltpu.sync_copy(bnd_h, bnd_v)
    pltpu.sync_copy(uid_h.at[tsl], uid_v)
    pltpu.sync_copy(g_h.at[tsl], gv_v)
    pltpu.sync_copy(s_h.at[tsl], sv_v)
    pltpu.sync_copy(rd_h.at[tsl], xb_v.at[0])
    pltpu.sync_copy(hy_h.at[tsl], xb_v.at[1])
    pltpu.sync_copy(vt_h.at[tsl], xb_v.at[2])
    pltpu.sync_copy(fv_h.at[tsl], xb_v.at[3])

    bidx = (bidx0, bidx1, bidx2, bidx3)

    def _precompute(g, carry):
        s16 = pl.ds(g * L, L)
        uq_v[s16] = uid_v[s16]
        gsidx_v[s16] = gv_v[s16] * 8 + sv_v[s16]
        for f in range(4):
            x = xb_v[f, s16]
            c = x * np.float32(NUM_BUCKETS - 1)
            t = c.astype(jnp.int32)
            e = t + jnp.where(t.astype(jnp.float32) < c, 1, 0)
            p = jnp.minimum(jnp.maximum(e - 2, 0), NUM_BUCKETS - 4)
            cnt = jnp.where(plsc.load_gather(bnd_v, [p]) < x, 1, 0)
            for k in range(1, 4):
                cnt = cnt + jnp.where(plsc.load_gather(bnd_v, [p + k]) < x, 1, 0)
            bidx[f][s16] = p + cnt
            nrm_v[f, s16] = (x - np.float32(0.5)) / np.float32(_DEN)
        return carry
    lax.fori_loop(0, B_PER_W // L, _precompute, 0)

    gsems = (sem_g0, sem_g1)
    uds = (ud0, ud1)
    gsds = (gsd0, gsd1)
    bkds = ((bk00, bk01, bk02, bk03), (bk10, bk11, bk12, bk13))
    stags = (stag0, stag1)
    osems = (sem_o0, sem_o1)

    def _fire(ci, s):
        csl = pl.ds(ci * CHUNK, CHUNK)
        sem = gsems[s]
        cps = [
            pltpu.async_copy(ut_h.at[uq_v.at[csl]], uds[s], sem),
            pltpu.async_copy(gs_h.at[gsidx_v.at[csl]], gsds[s], sem),
        ]
        for f in range(4):
            cps.append(
                pltpu.async_copy(bk_h.at[bidx[f].at[csl]], bkds[s][f], sem))
        return cps

    pend_g = {0: _fire(0, 0)}
    pend_o = {}
    for ci in range(N_CHUNKS):
        s = ci % 2
        if ci + 1 < N_CHUNKS:
            pend_g[ci + 1] = _fire(ci + 1, 1 - s)
        for cp in pend_g.pop(ci):
            cp.wait()
        if (ci - 2) in pend_o:
            pend_o.pop(ci - 2).wait()

        cb0 = ci * CHUNK
        ud_s, gsd_s, bkd_s, stag_s = uds[s], gsds[s], bkds[s], stags[s]

        def _repack(r, inner):
            rb = r * OUT_COLS
            stag_s[pl.ds(rb, L)] = ud_s[r, pl.ds(0, L)]
            stag_s[pl.ds(rb + L, L)] = ud_s[r, pl.ds(L, L)]
            for c in range(0, 2 * D, L):
                stag_s[pl.ds(rb + 32 + c, L)] = gsd_s[r, pl.ds(c, L)]
            for f in range(4):
                col = _FIELD_COLS[3 + f]
                for c in range(0, D, L):
                    stag_s[pl.ds(rb + col + c, L)] = \
                        bkd_s[f][r, pl.ds(f * D + c, L)]
            return inner
        lax.fori_loop(0, CHUNK, _repack, 0)

        for f in range(4):
            for j in range(CHUNK // L):
                rows = jnp.arange(j * L, (j + 1) * L, dtype=jnp.int32)
                pos = rows * OUT_COLS + _NORM_COLS[f]
                nx = nrm_v[f, pl.ds(cb0 + j * L, L)]
                plsc.store_scatter(stag_s, [pos], nx)

        cb = base + cb0
        pend_o[ci] = pltpu.async_copy(
            stag_s, out_h.at[pl.ds(cb * OUT_COLS, CHUNK * OUT_COLS)],
            osems[s])
    for cp in pend_o.values():
        cp.wait()


def kernel(user_id, gender, status, regis_date, history, voting, favourite,
           user_table, gender_table, status_table,
           rgst_table, hsty_table, vote_table, favr_table):
    bnd = jnp.linspace(0.0, 1.0, NUM_BUCKETS).astype(jnp.float32)
    bnd = jnp.concatenate([bnd, jnp.full((8,), 2.0, jnp.float32)])
    ut2 = jnp.pad(user_table, ((0, 0), (0, 3 * D)))
    gs = jnp.concatenate([jnp.repeat(gender_table, 8, axis=0),
                          jnp.tile(status_table, (3, 1)),
                          jnp.zeros((24, 2 * D), jnp.float32)], axis=1)
    bk = jnp.concatenate([rgst_table, hsty_table, vote_table, favr_table],
                         axis=1)
    mesh = plsc.VectorSubcoreMesh(core_axis_name="c", subcore_axis_name="s")
    gd = [pltpu.VMEM((CHUNK, 4 * D), jnp.float32)] * 12
    run = functools.partial(
        pl.kernel, mesh=mesh,
        compiler_params=pltpu.CompilerParams(needs_layout_passes=False),
        out_type=jax.ShapeDtypeStruct((B * OUT_COLS,), jnp.float32),
        scratch_types=[
            pltpu.VMEM((NUM_BUCKETS + 8,), jnp.float32),
            pltpu.VMEM((B_PER_W,), jnp.int32),
            pltpu.VMEM((B_PER_W,), jnp.int32),
            pltpu.VMEM((B_PER_W,), jnp.int32),
            pltpu.VMEM((B_PER_W,), jnp.int32),
            pltpu.VMEM((B_PER_W + L,), jnp.int32),
            pltpu.VMEM((B_PER_W,), jnp.int32),
            pltpu.VMEM((4, B_PER_W), jnp.float32),
            pltpu.VMEM((B_PER_W,), jnp.int32),
            pltpu.VMEM((B_PER_W,), jnp.int32),
            pltpu.VMEM((B_PER_W,), jnp.int32),
            pltpu.VMEM((B_PER_W,), jnp.int32),
            pltpu.VMEM((4, B_PER_W), jnp.float32),
        ] + gd + [
            pltpu.VMEM((CHUNK * OUT_COLS,), jnp.float32),
            pltpu.VMEM((CHUNK * OUT_COLS,), jnp.float32),
            pltpu.SemaphoreType.DMA,
            pltpu.SemaphoreType.DMA,
            pltpu.SemaphoreType.DMA,
            pltpu.SemaphoreType.DMA,
        ],
    )(_body)
    flat = run(user_id.astype(jnp.int32), gender.astype(jnp.int32),
               status.astype(jnp.int32), regis_date, history, voting,
               favourite, ut2, gs, bk, bnd)
    return flat.reshape(B, OUT_COLS)
